# Initial kernel scaffold; baseline (speedup 1.0000x reference)
import jax
import jax.numpy as jnp
from jax.experimental import pallas as pl


def kernel(fm, counts):
    raise NotImplementedError("write your pallas kernel here")


# speedup vs baseline: 1.5217x; 1.5217x over previous
import functools

import jax
import jax.numpy as jnp
from jax import lax
from jax.experimental import pallas as pl
from jax.experimental.pallas import tpu as pltpu
from jax.experimental.pallas import tpu_sc as plsc

B, C, H, W = 16, 384, 24, 24
P = H * W
K = 9
HP, WP = H + 2, W + 2
NW = 32
ROWS_PER_W = B * P * K // NW
CHUNK = 96
IDX_R, IDX_C = 8, P * K // 8


def _prep_body(fm_ref, cnt_ref, fmt_ref, idx_ref):
    fmt_ref[0] = fm_ref[0].T
    v = cnt_ref[...]
    i = lax.shift_right_logical(v * 5042, 17)
    j = v - i * WP
    ih = jnp.clip(i - 1, 0, H - 1)
    jw = jnp.clip(j - 1, 0, W - 1)
    idx_ref[...] = ih * W + jw


def _tc_prep(fm3, counts2):
    return pl.pallas_call(
        _prep_body,
        grid=(B,),
        in_specs=[
            pl.BlockSpec((1, C, P), lambda b: (b, 0, 0)),
            pl.BlockSpec((IDX_R, IDX_C), lambda b: (0, 0)),
        ],
        out_specs=[
            pl.BlockSpec((1, P, C), lambda b: (b, 0, 0)),
            pl.BlockSpec((IDX_R, IDX_C), lambda b: (0, 0)),
        ],
        out_shape=[
            jax.ShapeDtypeStruct((B, P, C), jnp.float32),
            jax.ShapeDtypeStruct((IDX_R, IDX_C), jnp.int32),
        ],
    )(fm3, counts2)


_SC_MESH = plsc.VectorSubcoreMesh(core_axis_name="c", subcore_axis_name="s")


@functools.partial(
    pl.kernel,
    mesh=_SC_MESH,
    out_type=jax.ShapeDtypeStruct((B * P * K, C), jnp.float32),
    scratch_types=[
        pltpu.VMEM((ROWS_PER_W,), jnp.int32),
        pltpu.VMEM((CHUNK, C), jnp.float32),
        pltpu.SemaphoreType.DMA,
    ],
)
def _sc_gather(table_hbm, idx_hbm, out_hbm, idx_v, rows_v, sem):
    wid = lax.axis_index("s") * 2 + lax.axis_index("c")
    batch = wid // 2
    half = wid % 2
    pltpu.sync_copy(idx_hbm.at[pl.ds(half * ROWS_PER_W, ROWS_PER_W)], idx_v)

    @pl.loop(0, ROWS_PER_W, step=16)
    def _(i):
        idx_v[pl.ds(i, 16)] = idx_v[pl.ds(i, 16)] + batch * P

    out_base = wid * ROWS_PER_W

    @pl.loop(0, ROWS_PER_W, step=CHUNK)
    def _(c):
        pltpu.async_copy(table_hbm.at[idx_v.at[pl.ds(c, CHUNK)]], rows_v,
                         sem).wait()
        pltpu.sync_copy(rows_v, out_hbm.at[pl.ds(out_base + c, CHUNK)])


def kernel(fm, counts):
    fm3 = fm.reshape(B, C, P)
    counts2 = counts.reshape(IDX_R, IDX_C)
    fmt, idxmap = _tc_prep(fm3, counts2)
    out = _sc_gather(fmt.reshape(B * P, C), idxmap.reshape(P * K))
    return out.reshape(B, P, K * C)


# speedup vs baseline: 1.5471x; 1.0167x over previous
import functools

import jax
import jax.numpy as jnp
from jax import lax
from jax.experimental import pallas as pl
from jax.experimental.pallas import tpu as pltpu
from jax.experimental.pallas import tpu_sc as plsc

B, C, H, W = 16, 384, 24, 24
P = H * W
K = 9
HP, WP = H + 2, W + 2
NW = 32
ROWS_PER_W = B * P * K // NW
CHUNK = 96
IDX_R, IDX_C = 8, P * K // 8


def _prep_body(fm_ref, cnt_ref, fmt_ref, idx_ref):
    fmt_ref[0] = fm_ref[0].T
    v = cnt_ref[...]
    i = lax.shift_right_logical(v * 5042, 17)
    j = v - i * WP
    ih = jnp.clip(i - 1, 0, H - 1)
    jw = jnp.clip(j - 1, 0, W - 1)
    idx_ref[...] = ih * W + jw


def _tc_prep(fm3, counts2):
    return pl.pallas_call(
        _prep_body,
        grid=(B,),
        in_specs=[
            pl.BlockSpec((1, C, P), lambda b: (b, 0, 0)),
            pl.BlockSpec((IDX_R, IDX_C), lambda b: (0, 0)),
        ],
        out_specs=[
            pl.BlockSpec((1, P, C), lambda b: (b, 0, 0)),
            pl.BlockSpec((IDX_R, IDX_C), lambda b: (0, 0)),
        ],
        out_shape=[
            jax.ShapeDtypeStruct((B, P, C), jnp.float32),
            jax.ShapeDtypeStruct((IDX_R, IDX_C), jnp.int32),
        ],
    )(fm3, counts2)


_SC_MESH = plsc.VectorSubcoreMesh(core_axis_name="c", subcore_axis_name="s")


@functools.partial(
    pl.kernel,
    mesh=_SC_MESH,
    out_type=jax.ShapeDtypeStruct((B * P * K, C), jnp.float32),
    scratch_types=[
        pltpu.VMEM((ROWS_PER_W,), jnp.int32),
        pltpu.VMEM((CHUNK, C), jnp.float32),
        pltpu.VMEM((CHUNK, C), jnp.float32),
        pltpu.SemaphoreType.DMA,
        pltpu.SemaphoreType.DMA,
    ],
)
def _sc_gather(table_hbm, idx_hbm, out_hbm, idx_v, buf0, buf1, sem0, sem1):
    wid = lax.axis_index("s") * 2 + lax.axis_index("c")
    batch = wid // 2
    half = wid % 2
    pltpu.sync_copy(idx_hbm.at[pl.ds(half * ROWS_PER_W, ROWS_PER_W)], idx_v)

    @pl.loop(0, ROWS_PER_W, step=16)
    def _(i):
        idx_v[pl.ds(i, 16)] = idx_v[pl.ds(i, 16)] + batch * P

    out_base = wid * ROWS_PER_W

    def start_gather(c, buf, sem):
        pltpu.async_copy(table_hbm.at[idx_v.at[pl.ds(c, CHUNK)]], buf, sem)

    def wait_gather(c, buf, sem):
        pltpu.make_async_copy(table_hbm.at[idx_v.at[pl.ds(c, CHUNK)]],
                              buf, sem).wait()

    start_gather(0, buf0, sem0)

    @pl.loop(0, ROWS_PER_W, step=2 * CHUNK)
    def _(c):
        @pl.when(c + CHUNK < ROWS_PER_W)
        def _():
            start_gather(c + CHUNK, buf1, sem1)

        wait_gather(c, buf0, sem0)
        pltpu.sync_copy(buf0, out_hbm.at[pl.ds(out_base + c, CHUNK)])

        @pl.when(c + 2 * CHUNK < ROWS_PER_W)
        def _():
            start_gather(c + 2 * CHUNK, buf0, sem0)

        @pl.when(c + CHUNK < ROWS_PER_W)
        def _():
            wait_gather(c + CHUNK, buf1, sem1)
            pltpu.sync_copy(buf1, out_hbm.at[pl.ds(out_base + c + CHUNK,
                                                   CHUNK)])


def kernel(fm, counts):
    fm3 = fm.reshape(B, C, P)
    counts2 = counts.reshape(IDX_R, IDX_C)
    fmt, idxmap = _tc_prep(fm3, counts2)
    out = _sc_gather(fmt.reshape(B * P, C), idxmap.reshape(P * K))
    return out.reshape(B, P, K * C)


# speedup vs baseline: 2.3154x; 1.4966x over previous
import functools

import jax
import jax.numpy as jnp
from jax import lax
from jax.experimental import pallas as pl
from jax.experimental.pallas import tpu as pltpu
from jax.experimental.pallas import tpu_sc as plsc

B, C, H, W = 16, 384, 24, 24
P = H * W
K = 9
HP, WP = H + 2, W + 2
NW = 32
POS_PER_W = P // 2
PCHUNK = 96


def _prep_body(fm_ref, cnt_ref, fmt_ref, idx_ref):
    fmt_ref[0] = fm_ref[0].reshape(C, P).T
    v = cnt_ref[...]
    i = lax.shift_right_logical(v * 5042, 17)
    j = v - i * WP
    ih = jnp.clip(i - 1, 0, H - 1)
    jw = jnp.clip(j - 1, 0, W - 1)
    t = (ih * W + jw).T
    idx_ref[0] = t[:, :POS_PER_W]
    idx_ref[1] = t[:, POS_PER_W:]


def _tc_prep(fm, counts):
    return pl.pallas_call(
        _prep_body,
        grid=(B,),
        in_specs=[
            pl.BlockSpec((1, C, H, W), lambda b: (b, 0, 0, 0)),
            pl.BlockSpec((P, K), lambda b: (0, 0)),
        ],
        out_specs=[
            pl.BlockSpec((1, P, C), lambda b: (b, 0, 0)),
            pl.BlockSpec((2, K, POS_PER_W), lambda b: (0, 0, 0)),
        ],
        out_shape=[
            jax.ShapeDtypeStruct((B, P, C), jnp.float32),
            jax.ShapeDtypeStruct((2, K, POS_PER_W), jnp.int32),
        ],
    )(fm, counts)


_SC_MESH = plsc.VectorSubcoreMesh(core_axis_name="c", subcore_axis_name="s")


@functools.partial(
    pl.kernel,
    mesh=_SC_MESH,
    out_type=jax.ShapeDtypeStruct((B * P, K * C), jnp.float32),
    scratch_types=[
        pltpu.VMEM((K * POS_PER_W,), jnp.int32),
        pltpu.VMEM((PCHUNK, C), jnp.float32),
        pltpu.VMEM((PCHUNK, C), jnp.float32),
        pltpu.SemaphoreType.DMA,
        pltpu.SemaphoreType.DMA,
    ],
)
def _sc_gather(table_hbm, idxk_hbm, out_hbm, idx_v, buf0, buf1, sem0, sem1):
    wid = lax.axis_index("s") * 2 + lax.axis_index("c")
    batch = wid // 2
    half = wid % 2
    pltpu.sync_copy(
        idxk_hbm.at[pl.ds(half * (K * POS_PER_W), K * POS_PER_W)], idx_v)

    @pl.loop(0, K * POS_PER_W, step=16)
    def _(i):
        idx_v[pl.ds(i, 16)] = idx_v[pl.ds(i, 16)] + batch * P

    pos0 = batch * P + half * POS_PER_W
    bufs = (buf0, buf1)
    sems = (sem0, sem1)

    def start_gather(k, c, j):
        pltpu.async_copy(
            table_hbm.at[idx_v.at[pl.ds(k * POS_PER_W + c, PCHUNK)]],
            bufs[j], sems[j])

    def wait_gather(k, c, j):
        pltpu.make_async_copy(
            table_hbm.at[idx_v.at[pl.ds(k * POS_PER_W + c, PCHUNK)]],
            bufs[j], sems[j]).wait()

    @pl.loop(0, POS_PER_W, step=PCHUNK)
    def _(c):
        start_gather(0, c, 0)
        for k in range(K):
            j = k % 2
            if k + 1 < K:
                start_gather(k + 1, c, 1 - j)
            wait_gather(k, c, j)
            pltpu.sync_copy(bufs[j],
                            out_hbm.at[pl.ds(pos0 + c, PCHUNK),
                                       pl.ds(k * C, C)])


def kernel(fm, counts):
    fmt, idxk = _tc_prep(fm, counts)
    out = _sc_gather(fmt.reshape(B * P, C), idxk.reshape(2 * K * POS_PER_W))
    return out.reshape(B, P, K * C)


# speedup vs baseline: 2.7603x; 1.1921x over previous
import functools

import jax
import jax.numpy as jnp
from jax import lax
from jax.experimental import pallas as pl
from jax.experimental.pallas import tpu as pltpu
from jax.experimental.pallas import tpu_sc as plsc

B, C, H, W = 16, 384, 24, 24
P = H * W
K = 9
HP, WP = H + 2, W + 2
NW = 32
POS_PER_W = P // 2
PCHUNK = 96


def _prep_body(fm_ref, cnt_ref, fmt_ref, idx_ref):
    fmt_ref[0] = fm_ref[0].T
    v = cnt_ref[...]
    i = lax.shift_right_logical(v * 5042, 17)
    j = v - i * WP
    ih = jnp.clip(i - 1, 0, H - 1)
    jw = jnp.clip(j - 1, 0, W - 1)
    t = (ih * W + jw).T
    idx_ref[0] = t[:, :POS_PER_W]
    idx_ref[1] = t[:, POS_PER_W:]


def _tc_prep(fm, counts):
    return pl.pallas_call(
        _prep_body,
        grid=(B,),
        in_specs=[
            pl.BlockSpec((1, C, P), lambda b: (b, 0, 0)),
            pl.BlockSpec((P, K), lambda b: (0, 0)),
        ],
        out_specs=[
            pl.BlockSpec((1, P, C), lambda b: (b, 0, 0)),
            pl.BlockSpec((2, K, POS_PER_W), lambda b: (0, 0, 0)),
        ],
        out_shape=[
            jax.ShapeDtypeStruct((B, P, C), jnp.float32),
            jax.ShapeDtypeStruct((2, K, POS_PER_W), jnp.int32),
        ],
    )(fm, counts)


_SC_MESH = plsc.VectorSubcoreMesh(core_axis_name="c", subcore_axis_name="s")


@functools.partial(
    pl.kernel,
    mesh=_SC_MESH,
    out_type=jax.ShapeDtypeStruct((B * P, K * C), jnp.float32),
    scratch_types=[
        pltpu.VMEM((K * POS_PER_W,), jnp.int32),
        pltpu.VMEM((PCHUNK, C), jnp.float32),
        pltpu.VMEM((PCHUNK, C), jnp.float32),
        pltpu.SemaphoreType.DMA,
        pltpu.SemaphoreType.DMA,
    ],
)
def _sc_gather(table_hbm, idxk_hbm, out_hbm, idx_v, buf0, buf1, sem0, sem1):
    wid = lax.axis_index("s") * 2 + lax.axis_index("c")
    batch = wid // 2
    half = wid % 2
    pltpu.sync_copy(
        idxk_hbm.at[pl.ds(half * (K * POS_PER_W), K * POS_PER_W)], idx_v)

    @pl.loop(0, K * POS_PER_W, step=16)
    def _(i):
        idx_v[pl.ds(i, 16)] = idx_v[pl.ds(i, 16)] + batch * P

    pos0 = batch * P + half * POS_PER_W
    bufs = (buf0, buf1)
    sems = (sem0, sem1)

    def start_gather(k, c, j):
        pltpu.async_copy(
            table_hbm.at[idx_v.at[pl.ds(k * POS_PER_W + c, PCHUNK)]],
            bufs[j], sems[j])

    def wait_gather(k, c, j):
        pltpu.make_async_copy(
            table_hbm.at[idx_v.at[pl.ds(k * POS_PER_W + c, PCHUNK)]],
            bufs[j], sems[j]).wait()

    @pl.loop(0, POS_PER_W, step=PCHUNK)
    def _(c):
        start_gather(0, c, 0)
        for k in range(K):
            j = k % 2
            if k + 1 < K:
                start_gather(k + 1, c, 1 - j)
            wait_gather(k, c, j)
            pltpu.sync_copy(bufs[j],
                            out_hbm.at[pl.ds(pos0 + c, PCHUNK),
                                       pl.ds(k * C, C)])


def kernel(fm, counts):
    fmt, idxk = _tc_prep(fm.reshape(B, C, P), counts)
    out = _sc_gather(fmt.reshape(B * P, C), idxk.reshape(2 * K * POS_PER_W))
    return out.reshape(B, P, K * C)


# speedup vs baseline: 2.8188x; 1.0212x over previous
import functools

import jax
import jax.numpy as jnp
from jax import lax
from jax.experimental import pallas as pl
from jax.experimental.pallas import tpu as pltpu
from jax.experimental.pallas import tpu_sc as plsc

B, C, H, W = 16, 384, 24, 24
P = H * W
K = 9
HP, WP = H + 2, W + 2
NW = 32
POS_PER_W = P // 2
PCHUNK = 72
NCHUNK = POS_PER_W // PCHUNK


def _prep_body(fm_ref, cnt_ref, fmt_ref, idx_ref):
    fmt_ref[0] = fm_ref[0].T
    v = cnt_ref[...]
    i = lax.shift_right_logical(v * 5042, 17)
    j = v - i * WP
    ih = jnp.clip(i - 1, 0, H - 1)
    jw = jnp.clip(j - 1, 0, W - 1)
    t = (ih * W + jw).T
    idx_ref[0] = t[:, :POS_PER_W]
    idx_ref[1] = t[:, POS_PER_W:]


def _tc_prep(fm, counts):
    return pl.pallas_call(
        _prep_body,
        grid=(B,),
        in_specs=[
            pl.BlockSpec((1, C, P), lambda b: (b, 0, 0)),
            pl.BlockSpec((P, K), lambda b: (0, 0)),
        ],
        out_specs=[
            pl.BlockSpec((1, P, C), lambda b: (b, 0, 0)),
            pl.BlockSpec((2, K, POS_PER_W), lambda b: (0, 0, 0)),
        ],
        out_shape=[
            jax.ShapeDtypeStruct((B, P, C), jnp.float32),
            jax.ShapeDtypeStruct((2, K, POS_PER_W), jnp.int32),
        ],
    )(fm, counts)


_SC_MESH = plsc.VectorSubcoreMesh(core_axis_name="c", subcore_axis_name="s")


@functools.partial(
    pl.kernel,
    mesh=_SC_MESH,
    out_type=jax.ShapeDtypeStruct((B * P, K * C), jnp.float32),
    scratch_types=[
        pltpu.VMEM((K * POS_PER_W,), jnp.int32),
        pltpu.VMEM((PCHUNK, C), jnp.float32),
        pltpu.VMEM((PCHUNK, C), jnp.float32),
        pltpu.VMEM((PCHUNK, C), jnp.float32),
        pltpu.VMEM((PCHUNK, C), jnp.float32),
        pltpu.SemaphoreType.DMA,
        pltpu.SemaphoreType.DMA,
        pltpu.SemaphoreType.DMA,
        pltpu.SemaphoreType.DMA,
        pltpu.SemaphoreType.DMA,
        pltpu.SemaphoreType.DMA,
        pltpu.SemaphoreType.DMA,
        pltpu.SemaphoreType.DMA,
    ],
)
def _sc_gather(table_hbm, idxk_hbm, out_hbm, idx_v, buf0, buf1, buf2, buf3,
               gs0, gs1, gs2, gs3, ss0, ss1, ss2, ss3):
    wid = lax.axis_index("s") * 2 + lax.axis_index("c")
    batch = wid // 2
    half = wid % 2
    pltpu.sync_copy(
        idxk_hbm.at[pl.ds(half * (K * POS_PER_W), K * POS_PER_W)], idx_v)

    @pl.loop(0, K * POS_PER_W, step=16)
    def _(i):
        idx_v[pl.ds(i, 16)] = idx_v[pl.ds(i, 16)] + batch * P

    pos0 = batch * P + half * POS_PER_W
    bufs = (buf0, buf1, buf2, buf3)
    gsems = (gs0, gs1, gs2, gs3)
    ssems = (ss0, ss1, ss2, ss3)

    def gather_copy(k, i):
        return pltpu.make_async_copy(
            table_hbm.at[idx_v.at[pl.ds(k * POS_PER_W + i * PCHUNK,
                                        PCHUNK)]],
            bufs[i], gsems[i])

    def store_copy(k, i):
        return pltpu.make_async_copy(
            bufs[i],
            out_hbm.at[pl.ds(pos0 + i * PCHUNK, PCHUNK),
                       pl.ds(k * C, C)],
            ssems[i])

    @pl.loop(0, K)
    def _(k):
        for i in range(NCHUNK):
            @pl.when(k > 0)
            def _(k=k, i=i):
                store_copy(k - 1, i).wait()
            gather_copy(k, i).start()
        for i in range(NCHUNK):
            gather_copy(k, i).wait()
            store_copy(k, i).start()

    for i in range(NCHUNK):
        store_copy(K - 1, i).wait()


def kernel(fm, counts):
    fmt, idxk = _tc_prep(fm.reshape(B, C, P), counts)
    out = _sc_gather(fmt.reshape(B * P, C), idxk.reshape(2 * K * POS_PER_W))
    return out.reshape(B, P, K * C)
